# baseline (speedup 1.0000x reference)
import functools

import jax
import jax.numpy as jnp
from jax import lax
from jax.experimental import pallas as pl
from jax.experimental.pallas import tpu as pltpu
from jax.experimental.pallas import tpu_sc as plsc

B = 16384
F = 26
VOCAB = 100000
FM_DIM = 8
EMB_DIM = 64
TOTAL = F * EMB_DIM
H1, H2 = 512, 256
EPS = 1e-5

NC, NS = 2, 16
NW = NC * NS
IDX_PER_DMA = 128
TOT_IDX = B * F
N_DMA = TOT_IDX // IDX_PER_DMA
DMA_PER_W = N_DMA // NW

BT = 512


def _sc_gather(xflat2d, deep_flat, fm_flat):
    mesh = plsc.VectorSubcoreMesh(core_axis_name="c", subcore_axis_name="s")

    @functools.partial(
        pl.kernel,
        mesh=mesh,
        compiler_params=pltpu.CompilerParams(use_tc_tiling_on_sc=False),
        out_type=[
            jax.ShapeDtypeStruct((TOT_IDX, EMB_DIM), jnp.bfloat16),
            jax.ShapeDtypeStruct((TOT_IDX, FM_DIM), jnp.float32),
        ],
        scratch_types=[
            pltpu.VMEM((DMA_PER_W, IDX_PER_DMA), jnp.int32),
            pltpu.VMEM((IDX_PER_DMA, EMB_DIM), jnp.bfloat16),
            pltpu.VMEM((IDX_PER_DMA, EMB_DIM), jnp.bfloat16),
            pltpu.VMEM((IDX_PER_DMA, FM_DIM), jnp.float32),
            pltpu.VMEM((IDX_PER_DMA, FM_DIM), jnp.float32),
            pltpu.SemaphoreType.DMA,
            pltpu.SemaphoreType.DMA,
            pltpu.SemaphoreType.DMA,
            pltpu.SemaphoreType.DMA,
        ],
    )
    def gather_kernel(x_hbm, deep_hbm, fm_hbm, deep_out, fm_out,
                      idx_v, d0, d1, f0, f1, gs0, gs1, ss0, ss1):
        wid = lax.axis_index("s") * NC + lax.axis_index("c")
        jbase = wid * DMA_PER_W
        pltpu.sync_copy(x_hbm.at[pl.ds(jbase, DMA_PER_W)], idx_v)
        dbuf, fbuf = (d0, d1), (f0, f1)
        gsem, ssem = (gs0, gs1), (ss0, ss1)

        def gathers(j, b):
            pltpu.async_copy(deep_hbm.at[idx_v.at[j]], dbuf[b], gsem[b])
            pltpu.async_copy(fm_hbm.at[idx_v.at[j]], fbuf[b], gsem[b])

        def wait_gathers(j, b):
            pltpu.make_async_copy(deep_hbm.at[idx_v.at[j]], dbuf[b],
                                  gsem[b]).wait()
            pltpu.make_async_copy(fm_hbm.at[idx_v.at[j]], fbuf[b],
                                  gsem[b]).wait()

        def stores(j, b):
            row0 = (jbase + j) * IDX_PER_DMA
            pltpu.async_copy(dbuf[b], deep_out.at[pl.ds(row0, IDX_PER_DMA)],
                             ssem[b])
            pltpu.async_copy(fbuf[b], fm_out.at[pl.ds(row0, IDX_PER_DMA)],
                             ssem[b])

        def wait_stores(j, b):
            row0 = (jbase + j) * IDX_PER_DMA
            pltpu.make_async_copy(dbuf[b],
                                  deep_out.at[pl.ds(row0, IDX_PER_DMA)],
                                  ssem[b]).wait()
            pltpu.make_async_copy(fbuf[b],
                                  fm_out.at[pl.ds(row0, IDX_PER_DMA)],
                                  ssem[b]).wait()

        gathers(0, 0)
        gathers(1, 1)

        def body(k, carry):
            for b in range(2):
                j = 2 * k + b
                wait_gathers(j, b)
                stores(j, b)
                wait_stores(j, b)

                @pl.when(j + 2 < DMA_PER_W)
                def _():
                    gathers(j + 2, b)
            return carry

        lax.fori_loop(0, DMA_PER_W // 2, body, 0)

    return gather_kernel(xflat2d, deep_flat, fm_flat)


def _mlp_layer(xin, wT, brow):
    bdim, kdim = xin.shape
    ndim = wT.shape[1]

    def body(x_ref, w_ref, b_ref, h_ref, s_ref, ss_ref):
        i = pl.program_id(0)
        h = jnp.dot(x_ref[...], w_ref[...], preferred_element_type=jnp.float32)
        h = jnp.maximum(h + b_ref[...], 0.0)
        h_ref[...] = h

        @pl.when(i == 0)
        def _():
            s_ref[...] = jnp.zeros_like(s_ref)
            ss_ref[...] = jnp.zeros_like(ss_ref)

        s_ref[...] += jnp.sum(h, axis=0, keepdims=True)
        ss_ref[...] += jnp.sum(h * h, axis=0, keepdims=True)

    return pl.pallas_call(
        body,
        grid=(bdim // BT,),
        in_specs=[
            pl.BlockSpec((BT, kdim), lambda i: (i, 0)),
            pl.BlockSpec((kdim, ndim), lambda i: (0, 0)),
            pl.BlockSpec((1, ndim), lambda i: (0, 0)),
        ],
        out_specs=[
            pl.BlockSpec((BT, ndim), lambda i: (i, 0)),
            pl.BlockSpec((1, ndim), lambda i: (0, 0)),
            pl.BlockSpec((1, ndim), lambda i: (0, 0)),
        ],
        out_shape=[
            jax.ShapeDtypeStruct((bdim, ndim), jnp.float32),
            jax.ShapeDtypeStruct((1, ndim), jnp.float32),
            jax.ShapeDtypeStruct((1, ndim), jnp.float32),
        ],
    )(xin, wT, brow)


def _final_layer(h2, fmc, w3row, b3p):
    def body(h_ref, fm_ref, w_ref, b_ref, o_ref):
        z = jnp.sum(h_ref[...] * w_ref[...], axis=1, keepdims=True) + b_ref[0]
        dp = jax.nn.sigmoid(z)
        f = fm_ref[...]
        r = lax.broadcasted_iota(jnp.int32, (F * FM_DIM, FM_DIM), 0)
        c = lax.broadcasted_iota(jnp.int32, (F * FM_DIM, FM_DIM), 1)
        m = (r % FM_DIM == c).astype(jnp.float32)
        s8 = jnp.dot(f, m, preferred_element_type=jnp.float32)
        fm = 0.5 * (jnp.sum(s8 * s8, axis=1, keepdims=True)
                    - jnp.sum(f * f, axis=1, keepdims=True))
        o_ref[...] = jnp.broadcast_to(dp + fm, (BT, FM_DIM))

    return pl.pallas_call(
        body,
        grid=(B // BT,),
        in_specs=[
            pl.BlockSpec((BT, H2), lambda i: (i, 0)),
            pl.BlockSpec((BT, F * FM_DIM), lambda i: (i, 0)),
            pl.BlockSpec((1, H2), lambda i: (0, 0)),
            pl.BlockSpec(memory_space=pltpu.SMEM),
        ],
        out_specs=pl.BlockSpec((BT, FM_DIM), lambda i: (i, 0)),
        out_shape=jax.ShapeDtypeStruct((B, FM_DIM), jnp.float32),
    )(h2, fmc, w3row, b3p)


def kernel(x, fm_tables, deep_tables, W1, b1, g1, be1, W2, b2, g2, be2, W3, b3):
    offs = (jnp.arange(F, dtype=jnp.int32) * VOCAB)[None, :]
    xflat2d = (x.astype(jnp.int32) + offs).reshape(N_DMA, IDX_PER_DMA)
    deep_flat = deep_tables.astype(jnp.bfloat16).reshape(F * VOCAB, EMB_DIM)
    fm_flat = fm_tables.reshape(F * VOCAB, FM_DIM)

    deep_rows, fm_rows = _sc_gather(xflat2d, deep_flat, fm_flat)
    dc = deep_rows.reshape(B, TOTAL)
    fmc = fm_rows.reshape(B, F * FM_DIM)

    h1, s1, ss1 = _mlp_layer(dc, W1.T.astype(jnp.bfloat16), b1[None, :])
    m1 = s1[0] / B
    v1 = ss1[0] / B - m1 * m1
    sc1 = g1 / jnp.sqrt(v1 + EPS)
    w2T = (W2 * sc1[None, :]).T
    b2p = b2 + W2 @ (be1 - m1 * sc1)

    h2, s2, ss2 = _mlp_layer(h1, w2T, b2p[None, :])
    m2 = s2[0] / B
    v2 = ss2[0] / B - m2 * m2
    sc2 = g2 / jnp.sqrt(v2 + EPS)
    w3row = (W3[0] * sc2)[None, :]
    b3p = b3 + W3[0] @ (be2 - m2 * sc2)

    res = _final_layer(h2, fmc, w3row, b3p)
    return res[:, 0]


# speedup vs baseline: 1.0730x; 1.0730x over previous
import functools

import jax
import jax.numpy as jnp
from jax import lax
from jax.experimental import pallas as pl
from jax.experimental.pallas import tpu as pltpu
from jax.experimental.pallas import tpu_sc as plsc

B = 16384
F = 26
VOCAB = 100000
FM_DIM = 8
EMB_DIM = 64
TOTAL = F * EMB_DIM
H1, H2 = 512, 256
EPS = 1e-5

NC, NS = 2, 16
NW = NC * NS
IDX_PER_DMA = 128
TOT_IDX = B * F
N_DMA = TOT_IDX // IDX_PER_DMA
DMA_PER_W = N_DMA // NW

BT = 512


def _sc_gather(xflat2d, deep_flat, fm_flat):
    mesh = plsc.VectorSubcoreMesh(core_axis_name="c", subcore_axis_name="s")

    @functools.partial(
        pl.kernel,
        mesh=mesh,
        compiler_params=pltpu.CompilerParams(use_tc_tiling_on_sc=False),
        out_type=[
            jax.ShapeDtypeStruct((TOT_IDX, EMB_DIM), jnp.float32),
            jax.ShapeDtypeStruct((TOT_IDX, FM_DIM), jnp.float32),
        ],
        scratch_types=[
            pltpu.VMEM((DMA_PER_W, IDX_PER_DMA), jnp.int32),
            pltpu.VMEM((IDX_PER_DMA, EMB_DIM), jnp.float32),
            pltpu.VMEM((IDX_PER_DMA, EMB_DIM), jnp.float32),
            pltpu.VMEM((IDX_PER_DMA, FM_DIM), jnp.float32),
            pltpu.VMEM((IDX_PER_DMA, FM_DIM), jnp.float32),
            pltpu.SemaphoreType.DMA,
            pltpu.SemaphoreType.DMA,
            pltpu.SemaphoreType.DMA,
            pltpu.SemaphoreType.DMA,
        ],
    )
    def gather_kernel(x_hbm, deep_hbm, fm_hbm, deep_out, fm_out,
                      idx_v, d0, d1, f0, f1, gs0, gs1, ss0, ss1):
        wid = lax.axis_index("s") * NC + lax.axis_index("c")
        jbase = wid * DMA_PER_W
        pltpu.sync_copy(x_hbm.at[pl.ds(jbase, DMA_PER_W)], idx_v)
        dbuf, fbuf = (d0, d1), (f0, f1)
        gsem, ssem = (gs0, gs1), (ss0, ss1)

        def gathers(j, b):
            pltpu.async_copy(deep_hbm.at[idx_v.at[j]], dbuf[b], gsem[b])
            pltpu.async_copy(fm_hbm.at[idx_v.at[j]], fbuf[b], gsem[b])

        def wait_gathers(j, b):
            pltpu.make_async_copy(deep_hbm.at[idx_v.at[j]], dbuf[b],
                                  gsem[b]).wait()
            pltpu.make_async_copy(fm_hbm.at[idx_v.at[j]], fbuf[b],
                                  gsem[b]).wait()

        def stores(j, b):
            row0 = (jbase + j) * IDX_PER_DMA
            pltpu.async_copy(dbuf[b], deep_out.at[pl.ds(row0, IDX_PER_DMA)],
                             ssem[b])
            pltpu.async_copy(fbuf[b], fm_out.at[pl.ds(row0, IDX_PER_DMA)],
                             ssem[b])

        def wait_stores(j, b):
            row0 = (jbase + j) * IDX_PER_DMA
            pltpu.make_async_copy(dbuf[b],
                                  deep_out.at[pl.ds(row0, IDX_PER_DMA)],
                                  ssem[b]).wait()
            pltpu.make_async_copy(fbuf[b],
                                  fm_out.at[pl.ds(row0, IDX_PER_DMA)],
                                  ssem[b]).wait()

        gathers(0, 0)
        gathers(1, 1)

        def body(k, carry):
            for b in range(2):
                j = 2 * k + b
                wait_gathers(j, b)
                stores(j, b)
                wait_stores(j, b)

                @pl.when(j + 2 < DMA_PER_W)
                def _():
                    gathers(j + 2, b)
            return carry

        lax.fori_loop(0, DMA_PER_W // 2, body, 0)

    return gather_kernel(xflat2d, deep_flat, fm_flat)


def _mlp_layer(xin, wT, brow):
    bdim, kdim = xin.shape
    ndim = wT.shape[1]

    def body(x_ref, w_ref, b_ref, h_ref, s_ref, ss_ref):
        i = pl.program_id(0)
        h = jnp.dot(x_ref[...], w_ref[...], preferred_element_type=jnp.float32)
        h = jnp.maximum(h + b_ref[...], 0.0)
        h_ref[...] = h

        @pl.when(i == 0)
        def _():
            s_ref[...] = jnp.zeros_like(s_ref)
            ss_ref[...] = jnp.zeros_like(ss_ref)

        s_ref[...] += jnp.sum(h, axis=0, keepdims=True)
        ss_ref[...] += jnp.sum(h * h, axis=0, keepdims=True)

    return pl.pallas_call(
        body,
        grid=(bdim // BT,),
        in_specs=[
            pl.BlockSpec((BT, kdim), lambda i: (i, 0)),
            pl.BlockSpec((kdim, ndim), lambda i: (0, 0)),
            pl.BlockSpec((1, ndim), lambda i: (0, 0)),
        ],
        out_specs=[
            pl.BlockSpec((BT, ndim), lambda i: (i, 0)),
            pl.BlockSpec((1, ndim), lambda i: (0, 0)),
            pl.BlockSpec((1, ndim), lambda i: (0, 0)),
        ],
        out_shape=[
            jax.ShapeDtypeStruct((bdim, ndim), jnp.float32),
            jax.ShapeDtypeStruct((1, ndim), jnp.float32),
            jax.ShapeDtypeStruct((1, ndim), jnp.float32),
        ],
    )(xin, wT, brow)


def _final_layer(h2, fmc, w3row, b3p):
    def body(h_ref, fm_ref, w_ref, b_ref, o_ref):
        z = jnp.sum(h_ref[...] * w_ref[...], axis=1, keepdims=True) + b_ref[0]
        dp = jax.nn.sigmoid(z)
        f = fm_ref[...]
        r = lax.broadcasted_iota(jnp.int32, (F * FM_DIM, FM_DIM), 0)
        c = lax.broadcasted_iota(jnp.int32, (F * FM_DIM, FM_DIM), 1)
        m = (r % FM_DIM == c).astype(jnp.float32)
        s8 = jnp.dot(f, m, preferred_element_type=jnp.float32)
        fm = 0.5 * (jnp.sum(s8 * s8, axis=1, keepdims=True)
                    - jnp.sum(f * f, axis=1, keepdims=True))
        o_ref[...] = jnp.broadcast_to(dp + fm, (BT, FM_DIM))

    return pl.pallas_call(
        body,
        grid=(B // BT,),
        in_specs=[
            pl.BlockSpec((BT, H2), lambda i: (i, 0)),
            pl.BlockSpec((BT, F * FM_DIM), lambda i: (i, 0)),
            pl.BlockSpec((1, H2), lambda i: (0, 0)),
            pl.BlockSpec(memory_space=pltpu.SMEM),
        ],
        out_specs=pl.BlockSpec((BT, FM_DIM), lambda i: (i, 0)),
        out_shape=jax.ShapeDtypeStruct((B, FM_DIM), jnp.float32),
    )(h2, fmc, w3row, b3p)


def kernel(x, fm_tables, deep_tables, W1, b1, g1, be1, W2, b2, g2, be2, W3, b3):
    offs = (jnp.arange(F, dtype=jnp.int32) * VOCAB)[None, :]
    xflat2d = (x.astype(jnp.int32) + offs).reshape(N_DMA, IDX_PER_DMA)
    one = 1.0 + 0.0 * jnp.sum(b3)
    deep_flat = (deep_tables * one).reshape(F * VOCAB, EMB_DIM)
    fm_flat = (fm_tables * one).reshape(F * VOCAB, FM_DIM)

    deep_rows, fm_rows = _sc_gather(xflat2d, deep_flat, fm_flat)
    dc = deep_rows.reshape(B, TOTAL)
    fmc = fm_rows.reshape(B, F * FM_DIM)

    h1, s1, ss1 = _mlp_layer(dc, W1.T, b1[None, :])
    m1 = s1[0] / B
    v1 = ss1[0] / B - m1 * m1
    sc1 = g1 / jnp.sqrt(v1 + EPS)
    w2T = (W2 * sc1[None, :]).T
    b2p = b2 + W2 @ (be1 - m1 * sc1)

    h2, s2, ss2 = _mlp_layer(h1, w2T, b2p[None, :])
    m2 = s2[0] / B
    v2 = ss2[0] / B - m2 * m2
    sc2 = g2 / jnp.sqrt(v2 + EPS)
    w3row = (W3[0] * sc2)[None, :]
    b3p = b3 + W3[0] @ (be2 - m2 * sc2)

    res = _final_layer(h2, fmc, w3row, b3p)
    return res[:, 0]


# speedup vs baseline: 1.2111x; 1.1287x over previous
import functools

import jax
import jax.numpy as jnp
from jax import lax
from jax.experimental import pallas as pl
from jax.experimental.pallas import tpu as pltpu
from jax.experimental.pallas import tpu_sc as plsc

B = 16384
F = 26
VOCAB = 100000
FM_DIM = 8
EMB_DIM = 64
TOTAL = F * EMB_DIM
H1, H2 = 512, 256
EPS = 1e-5

NC, NS = 2, 16
NW = NC * NS
IDX_PER_DMA = 128
TOT_IDX = B * F
N_DMA = TOT_IDX // IDX_PER_DMA
DMA_PER_W = N_DMA // NW

BT = 512


def _sc_gather(xflat2d, deep_flat, fm_flat):
    mesh = plsc.VectorSubcoreMesh(core_axis_name="c", subcore_axis_name="s")

    @functools.partial(
        pl.kernel,
        mesh=mesh,
        compiler_params=pltpu.CompilerParams(use_tc_tiling_on_sc=False),
        out_type=[
            jax.ShapeDtypeStruct((TOT_IDX, EMB_DIM), jnp.float32),
            jax.ShapeDtypeStruct((TOT_IDX, FM_DIM), jnp.float32),
        ],
        scratch_types=[
            pltpu.VMEM((DMA_PER_W, IDX_PER_DMA), jnp.int32),
            pltpu.VMEM((IDX_PER_DMA, EMB_DIM), jnp.float32),
            pltpu.VMEM((IDX_PER_DMA, EMB_DIM), jnp.float32),
            pltpu.VMEM((IDX_PER_DMA, FM_DIM), jnp.float32),
            pltpu.VMEM((IDX_PER_DMA, FM_DIM), jnp.float32),
            pltpu.SemaphoreType.DMA,
            pltpu.SemaphoreType.DMA,
            pltpu.SemaphoreType.DMA,
            pltpu.SemaphoreType.DMA,
        ],
    )
    def gather_kernel(x_hbm, deep_hbm, fm_hbm, deep_out, fm_out,
                      idx_v, d0, d1, f0, f1, gs0, gs1, ss0, ss1):
        wid = lax.axis_index("s") * NC + lax.axis_index("c")
        jbase = wid * DMA_PER_W
        pltpu.sync_copy(x_hbm.at[pl.ds(jbase, DMA_PER_W)], idx_v)
        dbuf, fbuf = (d0, d1), (f0, f1)
        gsem, ssem = (gs0, gs1), (ss0, ss1)

        def gathers(j, b):
            pltpu.async_copy(deep_hbm.at[idx_v.at[j]], dbuf[b], gsem[b])
            pltpu.async_copy(fm_hbm.at[idx_v.at[j]], fbuf[b], gsem[b])

        def wait_gathers(j, b):
            pltpu.make_async_copy(deep_hbm.at[idx_v.at[j]], dbuf[b],
                                  gsem[b]).wait()
            pltpu.make_async_copy(fm_hbm.at[idx_v.at[j]], fbuf[b],
                                  gsem[b]).wait()

        def stores(j, b):
            row0 = (jbase + j) * IDX_PER_DMA
            pltpu.async_copy(dbuf[b], deep_out.at[pl.ds(row0, IDX_PER_DMA)],
                             ssem[b])
            pltpu.async_copy(fbuf[b], fm_out.at[pl.ds(row0, IDX_PER_DMA)],
                             ssem[b])

        def wait_stores(j, b):
            row0 = (jbase + j) * IDX_PER_DMA
            pltpu.make_async_copy(dbuf[b],
                                  deep_out.at[pl.ds(row0, IDX_PER_DMA)],
                                  ssem[b]).wait()
            pltpu.make_async_copy(fbuf[b],
                                  fm_out.at[pl.ds(row0, IDX_PER_DMA)],
                                  ssem[b]).wait()

        gathers(0, 0)
        gathers(1, 1)

        def body(k, carry):
            for b in range(2):
                j = 2 * k + b
                wait_gathers(j, b)
                stores(j, b)
                wait_stores(j, b)

                @pl.when(j + 2 < DMA_PER_W)
                def _():
                    gathers(j + 2, b)
            return carry

        lax.fori_loop(0, DMA_PER_W // 2, body, 0)

    return gather_kernel(xflat2d, deep_flat, fm_flat)


def _mlp_layer(xin, wT, brow):
    bdim, kdim = xin.shape
    ndim = wT.shape[1]

    def body(x_ref, w_ref, b_ref, h_ref, s_ref, ss_ref):
        i = pl.program_id(0)
        h = jnp.dot(x_ref[...], w_ref[...], preferred_element_type=jnp.float32)
        h = jnp.maximum(h + b_ref[...], 0.0)
        h_ref[...] = h

        @pl.when(i == 0)
        def _():
            s_ref[...] = jnp.zeros_like(s_ref)
            ss_ref[...] = jnp.zeros_like(ss_ref)

        s_ref[...] += jnp.sum(h, axis=0, keepdims=True)
        ss_ref[...] += jnp.sum(h * h, axis=0, keepdims=True)

    return pl.pallas_call(
        body,
        grid=(bdim // BT,),
        in_specs=[
            pl.BlockSpec((BT, kdim), lambda i: (i, 0)),
            pl.BlockSpec((kdim, ndim), lambda i: (0, 0)),
            pl.BlockSpec((1, ndim), lambda i: (0, 0)),
        ],
        out_specs=[
            pl.BlockSpec((BT, ndim), lambda i: (i, 0)),
            pl.BlockSpec((1, ndim), lambda i: (0, 0)),
            pl.BlockSpec((1, ndim), lambda i: (0, 0)),
        ],
        out_shape=[
            jax.ShapeDtypeStruct((bdim, ndim), jnp.float32),
            jax.ShapeDtypeStruct((1, ndim), jnp.float32),
            jax.ShapeDtypeStruct((1, ndim), jnp.float32),
        ],
    )(xin, wT, brow)


def _final_layer(h2, fmc, w3row, b3p):
    def body(h_ref, fm_ref, w_ref, b_ref, o_ref):
        z = jnp.sum(h_ref[...] * w_ref[...], axis=1, keepdims=True) + b_ref[0]
        dp = jax.nn.sigmoid(z)
        f = fm_ref[...]
        r = lax.broadcasted_iota(jnp.int32, (F * FM_DIM, FM_DIM), 0)
        c = lax.broadcasted_iota(jnp.int32, (F * FM_DIM, FM_DIM), 1)
        m = (r % FM_DIM == c).astype(jnp.float32)
        s8 = jnp.dot(f, m, preferred_element_type=jnp.float32)
        fm = 0.5 * (jnp.sum(s8 * s8, axis=1, keepdims=True)
                    - jnp.sum(f * f, axis=1, keepdims=True))
        o_ref[...] = jnp.broadcast_to(dp + fm, (BT, FM_DIM))

    return pl.pallas_call(
        body,
        grid=(B // BT,),
        in_specs=[
            pl.BlockSpec((BT, H2), lambda i: (i, 0)),
            pl.BlockSpec((BT, F * FM_DIM), lambda i: (i, 0)),
            pl.BlockSpec((1, H2), lambda i: (0, 0)),
            pl.BlockSpec(memory_space=pltpu.SMEM),
        ],
        out_specs=pl.BlockSpec((BT, FM_DIM), lambda i: (i, 0)),
        out_shape=jax.ShapeDtypeStruct((B, FM_DIM), jnp.float32),
    )(h2, fmc, w3row, b3p)


def kernel(x, fm_tables, deep_tables, W1, b1, g1, be1, W2, b2, g2, be2, W3, b3):
    offs = (jnp.arange(F, dtype=jnp.int32) * VOCAB)[None, :]
    xflat2d = (x.astype(jnp.int32) + offs).reshape(N_DMA, IDX_PER_DMA)
    deep_flat = deep_tables.reshape(F * VOCAB, EMB_DIM)
    fm_flat = fm_tables.reshape(F * VOCAB, FM_DIM)

    deep_rows, fm_rows = _sc_gather(xflat2d, deep_flat, fm_flat)
    dc = deep_rows.reshape(B, TOTAL)
    fmc = fm_rows.reshape(B, F * FM_DIM)

    h1, s1, ss1 = _mlp_layer(dc, W1.T, b1[None, :])
    m1 = s1[0] / B
    v1 = ss1[0] / B - m1 * m1
    sc1 = g1 / jnp.sqrt(v1 + EPS)
    w2T = (W2 * sc1[None, :]).T
    b2p = b2 + W2 @ (be1 - m1 * sc1)

    h2, s2, ss2 = _mlp_layer(h1, w2T, b2p[None, :])
    m2 = s2[0] / B
    v2 = ss2[0] / B - m2 * m2
    sc2 = g2 / jnp.sqrt(v2 + EPS)
    w3row = (W3[0] * sc2)[None, :]
    b3p = b3 + W3[0] @ (be2 - m2 * sc2)

    res = _final_layer(h2, fmc, w3row, b3p)
    return res[:, 0]


# speedup vs baseline: 1.2237x; 1.0104x over previous
import functools

import jax
import jax.numpy as jnp
from jax import lax
from jax.experimental import pallas as pl
from jax.experimental.pallas import tpu as pltpu
from jax.experimental.pallas import tpu_sc as plsc

B = 16384
F = 26
VOCAB = 100000
FM_DIM = 8
EMB_DIM = 64
TOTAL = F * EMB_DIM
H1, H2 = 512, 256
EPS = 1e-5

NC, NS = 2, 16
NW = NC * NS
IDX_PER_DMA = 128
TOT_IDX = B * F
N_DMA = TOT_IDX // IDX_PER_DMA
DMA_PER_W = N_DMA // NW

BT = 512


def _sc_gather_one(width):
    mesh = plsc.VectorSubcoreMesh(core_axis_name="c", subcore_axis_name="s")

    @functools.partial(
        pl.kernel,
        mesh=mesh,
        compiler_params=pltpu.CompilerParams(use_tc_tiling_on_sc=False),
        out_type=jax.ShapeDtypeStruct((TOT_IDX, width), jnp.float32),
        scratch_types=[
            pltpu.VMEM((DMA_PER_W, IDX_PER_DMA), jnp.int32),
            pltpu.VMEM((IDX_PER_DMA, width), jnp.float32),
            pltpu.VMEM((IDX_PER_DMA, width), jnp.float32),
            pltpu.SemaphoreType.DMA,
            pltpu.SemaphoreType.DMA,
            pltpu.SemaphoreType.DMA,
            pltpu.SemaphoreType.DMA,
        ],
    )
    def gather_kernel(x_hbm, tab_hbm, out_hbm,
                      idx_v, d0, d1, gs0, gs1, ss0, ss1):
        wid = lax.axis_index("s") * NC + lax.axis_index("c")
        jbase = wid * DMA_PER_W
        pltpu.sync_copy(x_hbm.at[pl.ds(jbase, DMA_PER_W)], idx_v)
        dbuf = (d0, d1)
        gsem, ssem = (gs0, gs1), (ss0, ss1)

        def gather(j, b):
            pltpu.async_copy(tab_hbm.at[idx_v.at[j]], dbuf[b], gsem[b])

        def wait_gather(j, b):
            pltpu.make_async_copy(tab_hbm.at[idx_v.at[j]], dbuf[b],
                                  gsem[b]).wait()

        def store(j, b):
            row0 = (jbase + j) * IDX_PER_DMA
            pltpu.async_copy(dbuf[b], out_hbm.at[pl.ds(row0, IDX_PER_DMA)],
                             ssem[b])

        def wait_store(j, b):
            row0 = (jbase + j) * IDX_PER_DMA
            pltpu.make_async_copy(dbuf[b],
                                  out_hbm.at[pl.ds(row0, IDX_PER_DMA)],
                                  ssem[b]).wait()

        gather(0, 0)
        gather(1, 1)

        def body(k, carry):
            for b in range(2):
                j = 2 * k + b
                wait_gather(j, b)
                store(j, b)
                wait_store(j, b)

                @pl.when(j + 2 < DMA_PER_W)
                def _():
                    gather(j + 2, b)
            return carry

        lax.fori_loop(0, DMA_PER_W // 2, body, 0)

    return gather_kernel


def _mlp_layer(xin, wT, brow):
    bdim, kdim = xin.shape
    ndim = wT.shape[1]

    def body(x_ref, w_ref, b_ref, h_ref, s_ref, ss_ref):
        i = pl.program_id(0)
        h = jnp.dot(x_ref[...], w_ref[...], preferred_element_type=jnp.float32)
        h = jnp.maximum(h + b_ref[...], 0.0)
        h_ref[...] = h

        @pl.when(i == 0)
        def _():
            s_ref[...] = jnp.zeros_like(s_ref)
            ss_ref[...] = jnp.zeros_like(ss_ref)

        s_ref[...] += jnp.sum(h, axis=0, keepdims=True)
        ss_ref[...] += jnp.sum(h * h, axis=0, keepdims=True)

    return pl.pallas_call(
        body,
        grid=(bdim // BT,),
        in_specs=[
            pl.BlockSpec((BT, kdim), lambda i: (i, 0)),
            pl.BlockSpec((kdim, ndim), lambda i: (0, 0)),
            pl.BlockSpec((1, ndim), lambda i: (0, 0)),
        ],
        out_specs=[
            pl.BlockSpec((BT, ndim), lambda i: (i, 0)),
            pl.BlockSpec((1, ndim), lambda i: (0, 0)),
            pl.BlockSpec((1, ndim), lambda i: (0, 0)),
        ],
        out_shape=[
            jax.ShapeDtypeStruct((bdim, ndim), jnp.float32),
            jax.ShapeDtypeStruct((1, ndim), jnp.float32),
            jax.ShapeDtypeStruct((1, ndim), jnp.float32),
        ],
    )(xin, wT, brow)


def _final_layer(h2, fmc, w3row, b3p):
    def body(h_ref, fm_ref, w_ref, b_ref, o_ref):
        z = jnp.sum(h_ref[...] * w_ref[...], axis=1, keepdims=True) + b_ref[0]
        dp = jax.nn.sigmoid(z)
        f = fm_ref[...]
        r = lax.broadcasted_iota(jnp.int32, (F * FM_DIM, FM_DIM), 0)
        c = lax.broadcasted_iota(jnp.int32, (F * FM_DIM, FM_DIM), 1)
        m = (r % FM_DIM == c).astype(jnp.float32)
        s8 = jnp.dot(f, m, preferred_element_type=jnp.float32)
        fm = 0.5 * (jnp.sum(s8 * s8, axis=1, keepdims=True)
                    - jnp.sum(f * f, axis=1, keepdims=True))
        o_ref[...] = jnp.broadcast_to(dp + fm, (BT, FM_DIM))

    return pl.pallas_call(
        body,
        grid=(B // BT,),
        in_specs=[
            pl.BlockSpec((BT, H2), lambda i: (i, 0)),
            pl.BlockSpec((BT, F * FM_DIM), lambda i: (i, 0)),
            pl.BlockSpec((1, H2), lambda i: (0, 0)),
            pl.BlockSpec(memory_space=pltpu.SMEM),
        ],
        out_specs=pl.BlockSpec((BT, FM_DIM), lambda i: (i, 0)),
        out_shape=jax.ShapeDtypeStruct((B, FM_DIM), jnp.float32),
    )(h2, fmc, w3row, b3p)


def kernel(x, fm_tables, deep_tables, W1, b1, g1, be1, W2, b2, g2, be2, W3, b3):
    offs = (jnp.arange(F, dtype=jnp.int32) * VOCAB)[None, :]
    xflat2d = (x.astype(jnp.int32) + offs).reshape(N_DMA, IDX_PER_DMA)
    deep_flat = deep_tables.reshape(F * VOCAB, EMB_DIM)
    fm_flat = fm_tables.reshape(F * VOCAB, FM_DIM)

    deep_rows = _sc_gather_one(EMB_DIM)(xflat2d, deep_flat)
    fm_flat_gated = lax.optimization_barrier((fm_flat, deep_rows))[0]
    fm_rows = _sc_gather_one(FM_DIM)(xflat2d, fm_flat_gated)
    dc = deep_rows.reshape(B, TOTAL)
    fmc = fm_rows.reshape(B, F * FM_DIM)

    h1, s1, ss1 = _mlp_layer(dc, W1.T, b1[None, :])
    m1 = s1[0] / B
    v1 = ss1[0] / B - m1 * m1
    sc1 = g1 / jnp.sqrt(v1 + EPS)
    w2T = (W2 * sc1[None, :]).T
    b2p = b2 + W2 @ (be1 - m1 * sc1)

    h2, s2, ss2 = _mlp_layer(h1, w2T, b2p[None, :])
    m2 = s2[0] / B
    v2 = ss2[0] / B - m2 * m2
    sc2 = g2 / jnp.sqrt(v2 + EPS)
    w3row = (W3[0] * sc2)[None, :]
    b3p = b3 + W3[0] @ (be2 - m2 * sc2)

    res = _final_layer(h2, fmc, w3row, b3p)
    return res[:, 0]


# speedup vs baseline: 1.2361x; 1.0101x over previous
import functools

import jax
import jax.numpy as jnp
from jax import lax
from jax.experimental import pallas as pl
from jax.experimental.pallas import tpu as pltpu
from jax.experimental.pallas import tpu_sc as plsc

B = 16384
F = 26
VOCAB = 100000
FM_DIM = 8
EMB_DIM = 64
TOTAL = F * EMB_DIM
H1, H2 = 512, 256
EPS = 1e-5

NC, NS = 2, 16
NW = NC * NS
IDX_PER_DMA = 128
TOT_IDX = B * F
N_DMA = TOT_IDX // IDX_PER_DMA
DMA_PER_W = N_DMA // NW

BT = 1024


def _sc_gather_one(width):
    mesh = plsc.VectorSubcoreMesh(core_axis_name="c", subcore_axis_name="s")

    @functools.partial(
        pl.kernel,
        mesh=mesh,
        compiler_params=pltpu.CompilerParams(use_tc_tiling_on_sc=False),
        out_type=jax.ShapeDtypeStruct((TOT_IDX, width), jnp.float32),
        scratch_types=[
            pltpu.VMEM((DMA_PER_W, IDX_PER_DMA), jnp.int32),
            pltpu.VMEM((IDX_PER_DMA, width), jnp.float32),
            pltpu.VMEM((IDX_PER_DMA, width), jnp.float32),
            pltpu.SemaphoreType.DMA,
            pltpu.SemaphoreType.DMA,
            pltpu.SemaphoreType.DMA,
            pltpu.SemaphoreType.DMA,
        ],
    )
    def gather_kernel(x_hbm, tab_hbm, out_hbm,
                      idx_v, d0, d1, gs0, gs1, ss0, ss1):
        wid = lax.axis_index("s") * NC + lax.axis_index("c")
        jbase = wid * DMA_PER_W
        pltpu.sync_copy(x_hbm.at[pl.ds(jbase, DMA_PER_W)], idx_v)
        dbuf = (d0, d1)
        gsem, ssem = (gs0, gs1), (ss0, ss1)

        def gather(j, b):
            pltpu.async_copy(tab_hbm.at[idx_v.at[j]], dbuf[b], gsem[b])

        def wait_gather(j, b):
            pltpu.make_async_copy(tab_hbm.at[idx_v.at[j]], dbuf[b],
                                  gsem[b]).wait()

        def store(j, b):
            row0 = (jbase + j) * IDX_PER_DMA
            pltpu.async_copy(dbuf[b], out_hbm.at[pl.ds(row0, IDX_PER_DMA)],
                             ssem[b])

        def wait_store(j, b):
            row0 = (jbase + j) * IDX_PER_DMA
            pltpu.make_async_copy(dbuf[b],
                                  out_hbm.at[pl.ds(row0, IDX_PER_DMA)],
                                  ssem[b]).wait()

        gather(0, 0)
        gather(1, 1)

        def body(k, carry):
            for b in range(2):
                j = 2 * k + b
                wait_gather(j, b)
                store(j, b)
                wait_store(j, b)

                @pl.when(j + 2 < DMA_PER_W)
                def _():
                    gather(j + 2, b)
            return carry

        lax.fori_loop(0, DMA_PER_W // 2, body, 0)

    return gather_kernel


def _mlp_layer(xin, wT, brow):
    bdim, kdim = xin.shape
    ndim = wT.shape[1]

    def body(x_ref, w_ref, b_ref, h_ref, s_ref, ss_ref):
        i = pl.program_id(0)
        h = jnp.dot(x_ref[...], w_ref[...], preferred_element_type=jnp.float32)
        h = jnp.maximum(h + b_ref[...], 0.0)
        h_ref[...] = h

        @pl.when(i == 0)
        def _():
            s_ref[...] = jnp.zeros_like(s_ref)
            ss_ref[...] = jnp.zeros_like(ss_ref)

        s_ref[...] += jnp.sum(h, axis=0, keepdims=True)
        ss_ref[...] += jnp.sum(h * h, axis=0, keepdims=True)

    return pl.pallas_call(
        body,
        grid=(bdim // BT,),
        in_specs=[
            pl.BlockSpec((BT, kdim), lambda i: (i, 0)),
            pl.BlockSpec((kdim, ndim), lambda i: (0, 0)),
            pl.BlockSpec((1, ndim), lambda i: (0, 0)),
        ],
        out_specs=[
            pl.BlockSpec((BT, ndim), lambda i: (i, 0)),
            pl.BlockSpec((1, ndim), lambda i: (0, 0)),
            pl.BlockSpec((1, ndim), lambda i: (0, 0)),
        ],
        out_shape=[
            jax.ShapeDtypeStruct((bdim, ndim), jnp.float32),
            jax.ShapeDtypeStruct((1, ndim), jnp.float32),
            jax.ShapeDtypeStruct((1, ndim), jnp.float32),
        ],
    )(xin, wT, brow)


def _final_layer(h2, fmc, w3row, b3p):
    def body(h_ref, fm_ref, w_ref, b_ref, o_ref):
        z = jnp.sum(h_ref[...] * w_ref[...], axis=1, keepdims=True) + b_ref[0]
        dp = jax.nn.sigmoid(z)
        f = fm_ref[...]
        r = lax.broadcasted_iota(jnp.int32, (F * FM_DIM, FM_DIM), 0)
        c = lax.broadcasted_iota(jnp.int32, (F * FM_DIM, FM_DIM), 1)
        m = (r % FM_DIM == c).astype(jnp.float32)
        s8 = jnp.dot(f, m, preferred_element_type=jnp.float32)
        fm = 0.5 * (jnp.sum(s8 * s8, axis=1, keepdims=True)
                    - jnp.sum(f * f, axis=1, keepdims=True))
        o_ref[...] = jnp.broadcast_to(dp + fm, (BT, FM_DIM))

    return pl.pallas_call(
        body,
        grid=(B // BT,),
        in_specs=[
            pl.BlockSpec((BT, H2), lambda i: (i, 0)),
            pl.BlockSpec((BT, F * FM_DIM), lambda i: (i, 0)),
            pl.BlockSpec((1, H2), lambda i: (0, 0)),
            pl.BlockSpec(memory_space=pltpu.SMEM),
        ],
        out_specs=pl.BlockSpec((BT, FM_DIM), lambda i: (i, 0)),
        out_shape=jax.ShapeDtypeStruct((B, FM_DIM), jnp.float32),
    )(h2, fmc, w3row, b3p)


def kernel(x, fm_tables, deep_tables, W1, b1, g1, be1, W2, b2, g2, be2, W3, b3):
    offs = (jnp.arange(F, dtype=jnp.int32) * VOCAB)[None, :]
    xflat2d = (x.astype(jnp.int32) + offs).reshape(N_DMA, IDX_PER_DMA)
    deep_flat = deep_tables.reshape(F * VOCAB, EMB_DIM)
    fm_flat = fm_tables.reshape(F * VOCAB, FM_DIM)

    deep_rows = _sc_gather_one(EMB_DIM)(xflat2d, deep_flat)
    fm_flat_gated = lax.optimization_barrier((fm_flat, deep_rows))[0]
    fm_rows = _sc_gather_one(FM_DIM)(xflat2d, fm_flat_gated)
    dc = deep_rows.reshape(B, TOTAL)
    fmc = fm_rows.reshape(B, F * FM_DIM)

    h1, s1, ss1 = _mlp_layer(dc, W1.T, b1[None, :])
    m1 = s1[0] / B
    v1 = ss1[0] / B - m1 * m1
    sc1 = g1 / jnp.sqrt(v1 + EPS)
    w2T = (W2 * sc1[None, :]).T
    b2p = b2 + W2 @ (be1 - m1 * sc1)

    h2, s2, ss2 = _mlp_layer(h1, w2T, b2p[None, :])
    m2 = s2[0] / B
    v2 = ss2[0] / B - m2 * m2
    sc2 = g2 / jnp.sqrt(v2 + EPS)
    w3row = (W3[0] * sc2)[None, :]
    b3p = b3 + W3[0] @ (be2 - m2 * sc2)

    res = _final_layer(h2, fmc, w3row, b3p)
    return res[:, 0]


# speedup vs baseline: 1.2414x; 1.0043x over previous
import functools

import jax
import jax.numpy as jnp
from jax import lax
from jax.experimental import pallas as pl
from jax.experimental.pallas import tpu as pltpu
from jax.experimental.pallas import tpu_sc as plsc

B = 16384
F = 26
VOCAB = 100000
FM_DIM = 8
EMB_DIM = 64
TOTAL = F * EMB_DIM
H1, H2 = 512, 256
EPS = 1e-5

NC, NS = 2, 16
NW = NC * NS
IDX_PER_DMA = 128
TOT_IDX = B * F
N_DMA = TOT_IDX // IDX_PER_DMA
DMA_PER_W = N_DMA // NW

BT = 2048


def _sc_gather_one(width):
    mesh = plsc.VectorSubcoreMesh(core_axis_name="c", subcore_axis_name="s")

    @functools.partial(
        pl.kernel,
        mesh=mesh,
        compiler_params=pltpu.CompilerParams(use_tc_tiling_on_sc=False),
        out_type=jax.ShapeDtypeStruct((TOT_IDX, width), jnp.float32),
        scratch_types=[
            pltpu.VMEM((DMA_PER_W, IDX_PER_DMA), jnp.int32),
            pltpu.VMEM((IDX_PER_DMA, width), jnp.float32),
            pltpu.VMEM((IDX_PER_DMA, width), jnp.float32),
            pltpu.SemaphoreType.DMA,
            pltpu.SemaphoreType.DMA,
            pltpu.SemaphoreType.DMA,
            pltpu.SemaphoreType.DMA,
        ],
    )
    def gather_kernel(x_hbm, tab_hbm, out_hbm,
                      idx_v, d0, d1, gs0, gs1, ss0, ss1):
        wid = lax.axis_index("s") * NC + lax.axis_index("c")
        jbase = wid * DMA_PER_W
        pltpu.sync_copy(x_hbm.at[pl.ds(jbase, DMA_PER_W)], idx_v)
        dbuf = (d0, d1)
        gsem, ssem = (gs0, gs1), (ss0, ss1)

        def gather(j, b):
            pltpu.async_copy(tab_hbm.at[idx_v.at[j]], dbuf[b], gsem[b])

        def wait_gather(j, b):
            pltpu.make_async_copy(tab_hbm.at[idx_v.at[j]], dbuf[b],
                                  gsem[b]).wait()

        def store(j, b):
            row0 = (jbase + j) * IDX_PER_DMA
            pltpu.async_copy(dbuf[b], out_hbm.at[pl.ds(row0, IDX_PER_DMA)],
                             ssem[b])

        def wait_store(j, b):
            row0 = (jbase + j) * IDX_PER_DMA
            pltpu.make_async_copy(dbuf[b],
                                  out_hbm.at[pl.ds(row0, IDX_PER_DMA)],
                                  ssem[b]).wait()

        gather(0, 0)
        gather(1, 1)

        def body(k, carry):
            for b in range(2):
                j = 2 * k + b
                wait_gather(j, b)
                store(j, b)
                wait_store(j, b)

                @pl.when(j + 2 < DMA_PER_W)
                def _():
                    gather(j + 2, b)
            return carry

        lax.fori_loop(0, DMA_PER_W // 2, body, 0)

    return gather_kernel


def _mlp_layer(xin, wT, brow):
    bdim, kdim = xin.shape
    ndim = wT.shape[1]

    def body(x_ref, w_ref, b_ref, h_ref, s_ref, ss_ref):
        i = pl.program_id(0)
        h = jnp.dot(x_ref[...], w_ref[...], preferred_element_type=jnp.float32)
        h = jnp.maximum(h + b_ref[...], 0.0)
        h_ref[...] = h

        @pl.when(i == 0)
        def _():
            s_ref[...] = jnp.zeros_like(s_ref)
            ss_ref[...] = jnp.zeros_like(ss_ref)

        s_ref[...] += jnp.sum(h, axis=0, keepdims=True)
        ss_ref[...] += jnp.sum(h * h, axis=0, keepdims=True)

    return pl.pallas_call(
        body,
        grid=(bdim // BT,),
        in_specs=[
            pl.BlockSpec((BT, kdim), lambda i: (i, 0)),
            pl.BlockSpec((kdim, ndim), lambda i: (0, 0)),
            pl.BlockSpec((1, ndim), lambda i: (0, 0)),
        ],
        out_specs=[
            pl.BlockSpec((BT, ndim), lambda i: (i, 0)),
            pl.BlockSpec((1, ndim), lambda i: (0, 0)),
            pl.BlockSpec((1, ndim), lambda i: (0, 0)),
        ],
        out_shape=[
            jax.ShapeDtypeStruct((bdim, ndim), jnp.float32),
            jax.ShapeDtypeStruct((1, ndim), jnp.float32),
            jax.ShapeDtypeStruct((1, ndim), jnp.float32),
        ],
    )(xin, wT, brow)


def _final_layer(h2, fmc, w3row, b3p):
    def body(h_ref, fm_ref, w_ref, b_ref, o_ref):
        z = jnp.sum(h_ref[...] * w_ref[...], axis=1, keepdims=True) + b_ref[0]
        dp = jax.nn.sigmoid(z)
        f = fm_ref[...]
        r = lax.broadcasted_iota(jnp.int32, (F * FM_DIM, FM_DIM), 0)
        c = lax.broadcasted_iota(jnp.int32, (F * FM_DIM, FM_DIM), 1)
        m = (r % FM_DIM == c).astype(jnp.float32)
        s8 = jnp.dot(f, m, preferred_element_type=jnp.float32)
        fm = 0.5 * (jnp.sum(s8 * s8, axis=1, keepdims=True)
                    - jnp.sum(f * f, axis=1, keepdims=True))
        o_ref[...] = jnp.broadcast_to(dp + fm, (BT, FM_DIM))

    return pl.pallas_call(
        body,
        grid=(B // BT,),
        in_specs=[
            pl.BlockSpec((BT, H2), lambda i: (i, 0)),
            pl.BlockSpec((BT, F * FM_DIM), lambda i: (i, 0)),
            pl.BlockSpec((1, H2), lambda i: (0, 0)),
            pl.BlockSpec(memory_space=pltpu.SMEM),
        ],
        out_specs=pl.BlockSpec((BT, FM_DIM), lambda i: (i, 0)),
        out_shape=jax.ShapeDtypeStruct((B, FM_DIM), jnp.float32),
    )(h2, fmc, w3row, b3p)


def kernel(x, fm_tables, deep_tables, W1, b1, g1, be1, W2, b2, g2, be2, W3, b3):
    offs = (jnp.arange(F, dtype=jnp.int32) * VOCAB)[None, :]
    xflat2d = (x.astype(jnp.int32) + offs).reshape(N_DMA, IDX_PER_DMA)
    deep_flat = deep_tables.reshape(F * VOCAB, EMB_DIM)
    fm_flat = fm_tables.reshape(F * VOCAB, FM_DIM)

    deep_rows = _sc_gather_one(EMB_DIM)(xflat2d, deep_flat)
    fm_flat_gated = lax.optimization_barrier((fm_flat, deep_rows))[0]
    fm_rows = _sc_gather_one(FM_DIM)(xflat2d, fm_flat_gated)
    dc = deep_rows.reshape(B, TOTAL)
    fmc = fm_rows.reshape(B, F * FM_DIM)

    h1, s1, ss1 = _mlp_layer(dc, W1.T, b1[None, :])
    m1 = s1[0] / B
    v1 = ss1[0] / B - m1 * m1
    sc1 = g1 / jnp.sqrt(v1 + EPS)
    w2T = (W2 * sc1[None, :]).T
    b2p = b2 + W2 @ (be1 - m1 * sc1)

    h2, s2, ss2 = _mlp_layer(h1, w2T, b2p[None, :])
    m2 = s2[0] / B
    v2 = ss2[0] / B - m2 * m2
    sc2 = g2 / jnp.sqrt(v2 + EPS)
    w3row = (W3[0] * sc2)[None, :]
    b3p = b3 + W3[0] @ (be2 - m2 * sc2)

    res = _final_layer(h2, fmc, w3row, b3p)
    return res[:, 0]
